# baseline (speedup 1.0000x reference)
import jax
import jax.numpy as jnp
from jax import lax
from jax.experimental import pallas as pl
from jax.experimental.pallas import tpu as pltpu
from jax.experimental.pallas import tpu_sc as plsc

B, L, F, R = 32, 256, 128, 32
FR = F * R
LANES = 16
NC, NS = 2, 16
NW = NC * NS

BT = 32
NSC = B - BT
K = NW // NSC if NSC else 0
RW = R // K if K else 0
CL = 8
NCHUNK = L // CL
TBB = 4


def _sc_body(x_hbm, w1_hbm, w2_hbm, a1_hbm, a2_hbm,
             buf, wv1, wv2, acc1, acc2, sem0, sem1):
    wid = lax.axis_index("s") * NC + lax.axis_index("c")
    b = BT + wid // K
    r0 = (wid % K) * RW

    pltpu.sync_copy(w1_hbm.at[pl.ds(b * 2, 2)], wv1)
    pltpu.sync_copy(w2_hbm.at[pl.ds(b * 2, 2)], wv2)

    zero = jnp.zeros((LANES,), jnp.float32)

    @plsc.parallel_loop(0, RW * 8, step=1, unroll=4)
    def _zero_body(i):
        q = i >> 3
        c = (i & 7) * LANES
        acc1[q, pl.ds(c, LANES)] = zero
        acc2[q, pl.ds(c, LANES)] = zero

    sems = (sem0, sem1)

    def _chunk_copy(g, d):
        return pltpu.make_async_copy(
            x_hbm.at[b, pl.ds(g * CL, CL), pl.ds(r0, RW)], buf.at[d], sems[d])

    _chunk_copy(0, 0).start()

    def _compute(d, w1s, w2s):
        @plsc.parallel_loop(0, RW * 8, step=1, unroll=4)
        def _vbody(v):
            q = v >> 3
            c = (v & 7) * LANES
            xs = [buf[d, li, q, pl.ds(c, LANES)] for li in range(CL)]
            s1 = ((xs[0] * w1s[0] + xs[1] * w1s[1])
                  + (xs[2] * w1s[2] + xs[3] * w1s[3]))
            t1 = ((xs[4] * w1s[4] + xs[5] * w1s[5])
                  + (xs[6] * w1s[6] + xs[7] * w1s[7]))
            s2 = ((xs[0] * w2s[0] + xs[1] * w2s[1])
                  + (xs[2] * w2s[2] + xs[3] * w2s[3]))
            t2 = ((xs[4] * w2s[4] + xs[5] * w2s[5])
                  + (xs[6] * w2s[6] + xs[7] * w2s[7]))
            acc1[q, pl.ds(c, LANES)] = acc1[q, pl.ds(c, LANES)] + (s1 + t1)
            acc2[q, pl.ds(c, LANES)] = acc2[q, pl.ds(c, LANES)] + (s2 + t2)

    def _pair(gg, _):
        w1v = wv1[gg >> 3, pl.ds((gg & 7) * LANES, LANES)]
        w2v = wv2[gg >> 3, pl.ds((gg & 7) * LANES, LANES)]
        for d in range(2):
            g = gg * 2 + d
            w1s = [jnp.broadcast_to(w1v[d * CL + li], (LANES,))
                   for li in range(CL)]
            w2s = [jnp.broadcast_to(w2v[d * CL + li], (LANES,))
                   for li in range(CL)]

            @pl.when(g + 1 < NCHUNK)
            def _start_next():
                _chunk_copy(g + 1, 1 - d).start()

            _chunk_copy(g, d).wait()
            _compute(d, w1s, w2s)
        return 0

    lax.fori_loop(0, NCHUNK // 2, _pair, 0)

    out_row = (b - BT) * R + r0
    pltpu.sync_copy(acc1, a1_hbm.at[pl.ds(out_row, RW)])
    pltpu.sync_copy(acc2, a2_hbm.at[pl.ds(out_row, RW)])


@jax.jit
def _sc_reduce(x4, w1, w2):
    mesh = plsc.VectorSubcoreMesh(core_axis_name="c", subcore_axis_name="s",
                                  num_cores=NC, num_subcores=NS)
    return pl.kernel(
        _sc_body,
        out_type=(jax.ShapeDtypeStruct((NSC * R, 128), jnp.float32),
                  jax.ShapeDtypeStruct((NSC * R, 128), jnp.float32)),
        mesh=mesh,
        scratch_types=(
            pltpu.VMEM((2, CL, RW, 128), jnp.float32),
            pltpu.VMEM((2, 128), jnp.float32),
            pltpu.VMEM((2, 128), jnp.float32),
            pltpu.VMEM((RW, 128), jnp.float32),
            pltpu.VMEM((RW, 128), jnp.float32),
            pltpu.SemaphoreType.DMA,
            pltpu.SemaphoreType.DMA,
        ),
        name="cons_net_sc_reduce",
    )(x4, w1, w2)


def _tcr_body(w1, w2, x, y):
    b0 = pl.program_id(0) * TBB
    for j in range(TBB):
        xb = x[j]
        y1 = jnp.dot(w1[pl.ds(b0 + j, 1), :], xb,
                     preferred_element_type=jnp.float32)
        y2 = jnp.dot(w2[pl.ds(b0 + j, 1), :], xb,
                     preferred_element_type=jnp.float32)
        y[j] = jnp.concatenate([y1, y2], axis=0)


@jax.jit
def _tc_reduce(x3, w1, w2):
    return pl.pallas_call(
        _tcr_body,
        grid=(BT // TBB,),
        in_specs=[pl.BlockSpec((B, L), lambda i: (0, 0)),
                  pl.BlockSpec((B, L), lambda i: (0, 0)),
                  pl.BlockSpec((TBB, L, FR), lambda i: (i, 0, 0))],
        out_specs=pl.BlockSpec((TBB, 2, FR), lambda i: (i, 0, 0)),
        out_shape=jax.ShapeDtypeStruct((BT, 2, FR), jnp.float32),
        name="cons_net_tc_reduce",
    )(w1, w2, x3)


def _tc_body(y2, a1s, a2s, cl, cr, rf, rr, w1, w2, out, m1, m2):
    clv = cl[...]
    crv = cr[...]
    rrv = rr[...]

    def _bt(b, _):
        base = b * 2 * R
        acc = jnp.dot(clv, y2[pl.ds(base, R), :],
                      preferred_element_type=jnp.float32)
        acc = acc + jnp.dot(crv, y2[pl.ds(base + R, R), :],
                            preferred_element_type=jnp.float32)
        out[pl.ds(b * R, R), :] = acc + rrv * rf[pl.ds(b, 1), :]
        return 0

    def _bs(b, _):
        base = (b - BT) * R
        acc = jnp.dot(clv, a1s[pl.ds(base, R), :],
                      preferred_element_type=jnp.float32)
        acc = acc + jnp.dot(crv, a2s[pl.ds(base, R), :],
                            preferred_element_type=jnp.float32)
        out[pl.ds(b * R, R), :] = acc + rrv * rf[pl.ds(b, 1), :]
        return 0

    if BT:
        lax.fori_loop(0, BT, _bt, 0)
    if NSC:
        lax.fori_loop(BT, B, _bs, 0)
    m1[...] = jnp.max(w1[...], axis=1, keepdims=True)
    m2[...] = jnp.max(w2[...], axis=1, keepdims=True)


@jax.jit
def _tc_epilogue(y2, a1s, a2s, cl, cr, rf, rr, w1, w2):
    return pl.pallas_call(
        _tc_body,
        out_shape=(jax.ShapeDtypeStruct((B * R, F), jnp.float32),
                   jax.ShapeDtypeStruct((B, 1), jnp.float32),
                   jax.ShapeDtypeStruct((B, 1), jnp.float32)),
        name="cons_net_tc_epilogue",
    )(y2, a1s, a2s, cl, cr, rf, rr, w1, w2)


def kernel(x, arg1_weight, arg2_weight, root_filler, cons_l, cons_r, root_role):
    x4 = x.transpose(0, 1, 3, 2)
    x3 = x4.reshape(B, L, FR)
    w1_2d = arg1_weight.reshape(B * L // 128, 128)
    w2_2d = arg2_weight.reshape(B * L // 128, 128)

    if NSC:
        a1s, a2s = _sc_reduce(x4, w1_2d, w2_2d)
    else:
        a1s = a2s = jnp.zeros((1 * R, 128), jnp.float32)
    if BT:
        y = _tc_reduce(x3, arg1_weight, arg2_weight)
        y2 = y.reshape(BT * 2 * R, F)
    else:
        y2 = jnp.zeros((1 * 2 * R, F), jnp.float32)

    out_brf, m1, m2 = _tc_epilogue(
        y2, a1s, a2s, cons_l, cons_r,
        root_filler, root_role.reshape(R, 1),
        arg1_weight, arg2_weight)
    return (out_brf.reshape(B, R, F).transpose(0, 2, 1),
            m1.reshape(B), m2.reshape(B))


# speedup vs baseline: 2.8907x; 2.8907x over previous
import jax
import jax.numpy as jnp
from jax import lax
from jax.experimental import pallas as pl
from jax.experimental.pallas import tpu as pltpu
from jax.experimental.pallas import tpu_sc as plsc

B, L, F, R = 32, 256, 128, 32
FR = F * R
LANES = 16
NC, NS = 2, 16
NW = NC * NS

BT = 32
NSC = B - BT
K = NW // NSC if NSC else 0
RW = R // K if K else 0
CL = 8
NCHUNK = L // CL
TBB = 4


def _sc_body(x_hbm, w1_hbm, w2_hbm, a1_hbm, a2_hbm,
             buf, wv1, wv2, acc1, acc2, sem0, sem1):
    wid = lax.axis_index("s") * NC + lax.axis_index("c")
    b = BT + wid // K
    r0 = (wid % K) * RW

    pltpu.sync_copy(w1_hbm.at[pl.ds(b * 2, 2)], wv1)
    pltpu.sync_copy(w2_hbm.at[pl.ds(b * 2, 2)], wv2)

    zero = jnp.zeros((LANES,), jnp.float32)

    @plsc.parallel_loop(0, RW * 8, step=1, unroll=4)
    def _zero_body(i):
        q = i >> 3
        c = (i & 7) * LANES
        acc1[q, pl.ds(c, LANES)] = zero
        acc2[q, pl.ds(c, LANES)] = zero

    sems = (sem0, sem1)

    def _chunk_copy(g, d):
        return pltpu.make_async_copy(
            x_hbm.at[b, pl.ds(g * CL, CL), pl.ds(r0, RW)], buf.at[d], sems[d])

    _chunk_copy(0, 0).start()

    def _compute(d, w1s, w2s):
        @plsc.parallel_loop(0, RW * 8, step=1, unroll=4)
        def _vbody(v):
            q = v >> 3
            c = (v & 7) * LANES
            xs = [buf[d, li, q, pl.ds(c, LANES)] for li in range(CL)]
            s1 = ((xs[0] * w1s[0] + xs[1] * w1s[1])
                  + (xs[2] * w1s[2] + xs[3] * w1s[3]))
            t1 = ((xs[4] * w1s[4] + xs[5] * w1s[5])
                  + (xs[6] * w1s[6] + xs[7] * w1s[7]))
            s2 = ((xs[0] * w2s[0] + xs[1] * w2s[1])
                  + (xs[2] * w2s[2] + xs[3] * w2s[3]))
            t2 = ((xs[4] * w2s[4] + xs[5] * w2s[5])
                  + (xs[6] * w2s[6] + xs[7] * w2s[7]))
            acc1[q, pl.ds(c, LANES)] = acc1[q, pl.ds(c, LANES)] + (s1 + t1)
            acc2[q, pl.ds(c, LANES)] = acc2[q, pl.ds(c, LANES)] + (s2 + t2)

    def _pair(gg, _):
        w1v = wv1[gg >> 3, pl.ds((gg & 7) * LANES, LANES)]
        w2v = wv2[gg >> 3, pl.ds((gg & 7) * LANES, LANES)]
        for d in range(2):
            g = gg * 2 + d
            w1s = [jnp.broadcast_to(w1v[d * CL + li], (LANES,))
                   for li in range(CL)]
            w2s = [jnp.broadcast_to(w2v[d * CL + li], (LANES,))
                   for li in range(CL)]

            @pl.when(g + 1 < NCHUNK)
            def _start_next():
                _chunk_copy(g + 1, 1 - d).start()

            _chunk_copy(g, d).wait()
            _compute(d, w1s, w2s)
        return 0

    lax.fori_loop(0, NCHUNK // 2, _pair, 0)

    out_row = (b - BT) * R + r0
    pltpu.sync_copy(acc1, a1_hbm.at[pl.ds(out_row, RW)])
    pltpu.sync_copy(acc2, a2_hbm.at[pl.ds(out_row, RW)])


@jax.jit
def _sc_reduce(x4, w1, w2):
    mesh = plsc.VectorSubcoreMesh(core_axis_name="c", subcore_axis_name="s",
                                  num_cores=NC, num_subcores=NS)
    return pl.kernel(
        _sc_body,
        out_type=(jax.ShapeDtypeStruct((NSC * R, 128), jnp.float32),
                  jax.ShapeDtypeStruct((NSC * R, 128), jnp.float32)),
        mesh=mesh,
        scratch_types=(
            pltpu.VMEM((2, CL, RW, 128), jnp.float32),
            pltpu.VMEM((2, 128), jnp.float32),
            pltpu.VMEM((2, 128), jnp.float32),
            pltpu.VMEM((RW, 128), jnp.float32),
            pltpu.VMEM((RW, 128), jnp.float32),
            pltpu.SemaphoreType.DMA,
            pltpu.SemaphoreType.DMA,
        ),
        name="cons_net_sc_reduce",
    )(x4, w1, w2)


CLT = 64


def _tcr_body(w1, w2, x, y):
    b0 = pl.program_id(0) * TBB
    lc = pl.program_id(1)
    l0 = lc * CLT

    zero = jnp.zeros((R, F), jnp.float32)
    init = []
    for j in range(TBB):
        init.append(jnp.where(lc > 0, y[j, 0], zero))
        init.append(jnp.where(lc > 0, y[j, 1], zero))

    def _lbody(l, accs):
        new = []
        for j in range(TBB):
            xl = x[j, l]
            new.append(accs[2 * j] + w1[b0 + j, l0 + l] * xl)
            new.append(accs[2 * j + 1] + w2[b0 + j, l0 + l] * xl)
        return tuple(new)

    accs = lax.fori_loop(0, CLT, _lbody, tuple(init), unroll=4)
    for j in range(TBB):
        y[j, 0] = accs[2 * j]
        y[j, 1] = accs[2 * j + 1]


@jax.jit
def _tc_reduce(x4, w1, w2):
    smem_full = pl.BlockSpec((B, L), lambda i, lc: (0, 0),
                             memory_space=pltpu.SMEM)
    return pl.pallas_call(
        _tcr_body,
        grid=(BT // TBB, L // CLT),
        in_specs=[smem_full, smem_full,
                  pl.BlockSpec((TBB, CLT, R, F), lambda i, lc: (i, lc, 0, 0))],
        out_specs=pl.BlockSpec((TBB, 2, R, F), lambda i, lc: (i, 0, 0, 0)),
        out_shape=jax.ShapeDtypeStruct((BT, 2, R, F), jnp.float32),
        name="cons_net_tc_reduce",
    )(w1, w2, x4)


def _tc_body(y2, a1s, a2s, cl, cr, rf, rr, w1, w2, out, m1, m2):
    clv = cl[...]
    crv = cr[...]
    rrv = rr[...]

    def _bt(b, _):
        base = b * 2 * R
        acc = jnp.dot(clv, y2[pl.ds(base, R), :],
                      preferred_element_type=jnp.float32)
        acc = acc + jnp.dot(crv, y2[pl.ds(base + R, R), :],
                            preferred_element_type=jnp.float32)
        out[pl.ds(b * R, R), :] = acc + rrv * rf[pl.ds(b, 1), :]
        return 0

    def _bs(b, _):
        base = (b - BT) * R
        acc = jnp.dot(clv, a1s[pl.ds(base, R), :],
                      preferred_element_type=jnp.float32)
        acc = acc + jnp.dot(crv, a2s[pl.ds(base, R), :],
                            preferred_element_type=jnp.float32)
        out[pl.ds(b * R, R), :] = acc + rrv * rf[pl.ds(b, 1), :]
        return 0

    if BT:
        lax.fori_loop(0, BT, _bt, 0)
    if NSC:
        lax.fori_loop(BT, B, _bs, 0)
    m1[...] = jnp.max(w1[...], axis=1, keepdims=True)
    m2[...] = jnp.max(w2[...], axis=1, keepdims=True)


@jax.jit
def _tc_epilogue(y2, a1s, a2s, cl, cr, rf, rr, w1, w2):
    return pl.pallas_call(
        _tc_body,
        out_shape=(jax.ShapeDtypeStruct((B * R, F), jnp.float32),
                   jax.ShapeDtypeStruct((B, 1), jnp.float32),
                   jax.ShapeDtypeStruct((B, 1), jnp.float32)),
        name="cons_net_tc_epilogue",
    )(y2, a1s, a2s, cl, cr, rf, rr, w1, w2)


def kernel(x, arg1_weight, arg2_weight, root_filler, cons_l, cons_r, root_role):
    x4 = x.transpose(0, 1, 3, 2)
    x3 = x4.reshape(B, L, FR)
    w1_2d = arg1_weight.reshape(B * L // 128, 128)
    w2_2d = arg2_weight.reshape(B * L // 128, 128)

    if NSC:
        a1s, a2s = _sc_reduce(x4, w1_2d, w2_2d)
    else:
        a1s = a2s = jnp.zeros((1 * R, 128), jnp.float32)
    if BT:
        y = _tc_reduce(x4, arg1_weight, arg2_weight)
        y2 = y.reshape(BT * 2 * R, F)
    else:
        y2 = jnp.zeros((1 * 2 * R, F), jnp.float32)

    out_brf, m1, m2 = _tc_epilogue(
        y2, a1s, a2s, cons_l, cons_r,
        root_filler, root_role.reshape(R, 1),
        arg1_weight, arg2_weight)
    return (out_brf.reshape(B, R, F).transpose(0, 2, 1),
            m1.reshape(B), m2.reshape(B))


# speedup vs baseline: 3.2889x; 1.1378x over previous
import jax
import jax.numpy as jnp
from jax import lax
from jax.experimental import pallas as pl
from jax.experimental.pallas import tpu as pltpu
from jax.experimental.pallas import tpu_sc as plsc

B, L, F, R = 32, 256, 128, 32
FR = F * R
LANES = 16
NC, NS = 2, 16
NW = NC * NS

BT = 32
NSC = B - BT
K = NW // NSC if NSC else 0
RW = R // K if K else 0
CL = 8
NCHUNK = L // CL
TBB = 4


def _sc_body(x_hbm, w1_hbm, w2_hbm, a1_hbm, a2_hbm,
             buf, wv1, wv2, acc1, acc2, sem0, sem1):
    wid = lax.axis_index("s") * NC + lax.axis_index("c")
    b = BT + wid // K
    r0 = (wid % K) * RW

    pltpu.sync_copy(w1_hbm.at[pl.ds(b * 2, 2)], wv1)
    pltpu.sync_copy(w2_hbm.at[pl.ds(b * 2, 2)], wv2)

    zero = jnp.zeros((LANES,), jnp.float32)

    @plsc.parallel_loop(0, RW * 8, step=1, unroll=4)
    def _zero_body(i):
        q = i >> 3
        c = (i & 7) * LANES
        acc1[q, pl.ds(c, LANES)] = zero
        acc2[q, pl.ds(c, LANES)] = zero

    sems = (sem0, sem1)

    def _chunk_copy(g, d):
        return pltpu.make_async_copy(
            x_hbm.at[b, pl.ds(g * CL, CL), pl.ds(r0, RW)], buf.at[d], sems[d])

    _chunk_copy(0, 0).start()

    def _compute(d, w1s, w2s):
        @plsc.parallel_loop(0, RW * 8, step=1, unroll=4)
        def _vbody(v):
            q = v >> 3
            c = (v & 7) * LANES
            xs = [buf[d, li, q, pl.ds(c, LANES)] for li in range(CL)]
            s1 = ((xs[0] * w1s[0] + xs[1] * w1s[1])
                  + (xs[2] * w1s[2] + xs[3] * w1s[3]))
            t1 = ((xs[4] * w1s[4] + xs[5] * w1s[5])
                  + (xs[6] * w1s[6] + xs[7] * w1s[7]))
            s2 = ((xs[0] * w2s[0] + xs[1] * w2s[1])
                  + (xs[2] * w2s[2] + xs[3] * w2s[3]))
            t2 = ((xs[4] * w2s[4] + xs[5] * w2s[5])
                  + (xs[6] * w2s[6] + xs[7] * w2s[7]))
            acc1[q, pl.ds(c, LANES)] = acc1[q, pl.ds(c, LANES)] + (s1 + t1)
            acc2[q, pl.ds(c, LANES)] = acc2[q, pl.ds(c, LANES)] + (s2 + t2)

    def _pair(gg, _):
        w1v = wv1[gg >> 3, pl.ds((gg & 7) * LANES, LANES)]
        w2v = wv2[gg >> 3, pl.ds((gg & 7) * LANES, LANES)]
        for d in range(2):
            g = gg * 2 + d
            w1s = [jnp.broadcast_to(w1v[d * CL + li], (LANES,))
                   for li in range(CL)]
            w2s = [jnp.broadcast_to(w2v[d * CL + li], (LANES,))
                   for li in range(CL)]

            @pl.when(g + 1 < NCHUNK)
            def _start_next():
                _chunk_copy(g + 1, 1 - d).start()

            _chunk_copy(g, d).wait()
            _compute(d, w1s, w2s)
        return 0

    lax.fori_loop(0, NCHUNK // 2, _pair, 0)

    out_row = (b - BT) * R + r0
    pltpu.sync_copy(acc1, a1_hbm.at[pl.ds(out_row, RW)])
    pltpu.sync_copy(acc2, a2_hbm.at[pl.ds(out_row, RW)])


@jax.jit
def _sc_reduce(x4, w1, w2):
    mesh = plsc.VectorSubcoreMesh(core_axis_name="c", subcore_axis_name="s",
                                  num_cores=NC, num_subcores=NS)
    return pl.kernel(
        _sc_body,
        out_type=(jax.ShapeDtypeStruct((NSC * R, 128), jnp.float32),
                  jax.ShapeDtypeStruct((NSC * R, 128), jnp.float32)),
        mesh=mesh,
        scratch_types=(
            pltpu.VMEM((2, CL, RW, 128), jnp.float32),
            pltpu.VMEM((2, 128), jnp.float32),
            pltpu.VMEM((2, 128), jnp.float32),
            pltpu.VMEM((RW, 128), jnp.float32),
            pltpu.VMEM((RW, 128), jnp.float32),
            pltpu.SemaphoreType.DMA,
            pltpu.SemaphoreType.DMA,
        ),
        name="cons_net_sc_reduce",
    )(x4, w1, w2)


CLT = 128


def _tcr_body(w1, w2, x, y):
    b0 = pl.program_id(0) * TBB
    lc = pl.program_id(1)
    l0 = lc * CLT

    zero = jnp.zeros((R, F), jnp.float32)
    init = []
    for j in range(TBB):
        init.append(jnp.where(lc > 0, y[j, 0], zero))
        init.append(jnp.where(lc > 0, y[j, 1], zero))

    def _lbody(l, accs):
        new = []
        for j in range(TBB):
            xl = x[j, l]
            new.append(accs[2 * j] + w1[b0 + j, l0 + l] * xl)
            new.append(accs[2 * j + 1] + w2[b0 + j, l0 + l] * xl)
        return tuple(new)

    accs = lax.fori_loop(0, CLT, _lbody, tuple(init), unroll=4)
    for j in range(TBB):
        y[j, 0] = accs[2 * j]
        y[j, 1] = accs[2 * j + 1]


@jax.jit
def _tc_reduce(x4, w1, w2):
    smem_full = pl.BlockSpec((B, L), lambda i, lc: (0, 0),
                             memory_space=pltpu.SMEM)
    return pl.pallas_call(
        _tcr_body,
        grid=(BT // TBB, L // CLT),
        in_specs=[smem_full, smem_full,
                  pl.BlockSpec((TBB, CLT, R, F), lambda i, lc: (i, lc, 0, 0))],
        out_specs=pl.BlockSpec((TBB, 2, R, F), lambda i, lc: (i, 0, 0, 0)),
        out_shape=jax.ShapeDtypeStruct((BT, 2, R, F), jnp.float32),
        name="cons_net_tc_reduce",
    )(w1, w2, x4)


def _tc_body(y2, a1s, a2s, cl, cr, rf, rr, w1, w2, out, m1, m2):
    clv = cl[...]
    crv = cr[...]
    rrv = rr[...]

    def _bt(b, _):
        base = b * 2 * R
        acc = jnp.dot(clv, y2[pl.ds(base, R), :],
                      preferred_element_type=jnp.float32)
        acc = acc + jnp.dot(crv, y2[pl.ds(base + R, R), :],
                            preferred_element_type=jnp.float32)
        out[pl.ds(b * R, R), :] = acc + rrv * rf[pl.ds(b, 1), :]
        return 0

    def _bs(b, _):
        base = (b - BT) * R
        acc = jnp.dot(clv, a1s[pl.ds(base, R), :],
                      preferred_element_type=jnp.float32)
        acc = acc + jnp.dot(crv, a2s[pl.ds(base, R), :],
                            preferred_element_type=jnp.float32)
        out[pl.ds(b * R, R), :] = acc + rrv * rf[pl.ds(b, 1), :]
        return 0

    if BT:
        lax.fori_loop(0, BT, _bt, 0)
    if NSC:
        lax.fori_loop(BT, B, _bs, 0)
    m1[...] = jnp.max(w1[...], axis=1, keepdims=True)
    m2[...] = jnp.max(w2[...], axis=1, keepdims=True)


@jax.jit
def _tc_epilogue(y2, a1s, a2s, cl, cr, rf, rr, w1, w2):
    return pl.pallas_call(
        _tc_body,
        out_shape=(jax.ShapeDtypeStruct((B * R, F), jnp.float32),
                   jax.ShapeDtypeStruct((B, 1), jnp.float32),
                   jax.ShapeDtypeStruct((B, 1), jnp.float32)),
        name="cons_net_tc_epilogue",
    )(y2, a1s, a2s, cl, cr, rf, rr, w1, w2)


def kernel(x, arg1_weight, arg2_weight, root_filler, cons_l, cons_r, root_role):
    x4 = x.transpose(0, 1, 3, 2)
    x3 = x4.reshape(B, L, FR)
    w1_2d = arg1_weight.reshape(B * L // 128, 128)
    w2_2d = arg2_weight.reshape(B * L // 128, 128)

    if NSC:
        a1s, a2s = _sc_reduce(x4, w1_2d, w2_2d)
    else:
        a1s = a2s = jnp.zeros((1 * R, 128), jnp.float32)
    if BT:
        y = _tc_reduce(x4, arg1_weight, arg2_weight)
        y2 = y.reshape(BT * 2 * R, F)
    else:
        y2 = jnp.zeros((1 * 2 * R, F), jnp.float32)

    out_brf, m1, m2 = _tc_epilogue(
        y2, a1s, a2s, cons_l, cons_r,
        root_filler, root_role.reshape(R, 1),
        arg1_weight, arg2_weight)
    return (out_brf.reshape(B, R, F).transpose(0, 2, 1),
            m1.reshape(B), m2.reshape(B))


# speedup vs baseline: 3.3525x; 1.0193x over previous
import jax
import jax.numpy as jnp
from jax import lax
from jax.experimental import pallas as pl
from jax.experimental.pallas import tpu as pltpu
from jax.experimental.pallas import tpu_sc as plsc

B, L, F, R = 32, 256, 128, 32
FR = F * R
LANES = 16
NC, NS = 2, 16
NW = NC * NS

BT = 32
NSC = B - BT
K = NW // NSC if NSC else 0
RW = R // K if K else 0
CL = 8
NCHUNK = L // CL
TBB = 4


def _sc_body(x_hbm, w1_hbm, w2_hbm, a1_hbm, a2_hbm,
             buf, wv1, wv2, acc1, acc2, sem0, sem1):
    wid = lax.axis_index("s") * NC + lax.axis_index("c")
    b = BT + wid // K
    r0 = (wid % K) * RW

    pltpu.sync_copy(w1_hbm.at[pl.ds(b * 2, 2)], wv1)
    pltpu.sync_copy(w2_hbm.at[pl.ds(b * 2, 2)], wv2)

    zero = jnp.zeros((LANES,), jnp.float32)

    @plsc.parallel_loop(0, RW * 8, step=1, unroll=4)
    def _zero_body(i):
        q = i >> 3
        c = (i & 7) * LANES
        acc1[q, pl.ds(c, LANES)] = zero
        acc2[q, pl.ds(c, LANES)] = zero

    sems = (sem0, sem1)

    def _chunk_copy(g, d):
        return pltpu.make_async_copy(
            x_hbm.at[b, pl.ds(g * CL, CL), pl.ds(r0, RW)], buf.at[d], sems[d])

    _chunk_copy(0, 0).start()

    def _compute(d, w1s, w2s):
        @plsc.parallel_loop(0, RW * 8, step=1, unroll=4)
        def _vbody(v):
            q = v >> 3
            c = (v & 7) * LANES
            xs = [buf[d, li, q, pl.ds(c, LANES)] for li in range(CL)]
            s1 = ((xs[0] * w1s[0] + xs[1] * w1s[1])
                  + (xs[2] * w1s[2] + xs[3] * w1s[3]))
            t1 = ((xs[4] * w1s[4] + xs[5] * w1s[5])
                  + (xs[6] * w1s[6] + xs[7] * w1s[7]))
            s2 = ((xs[0] * w2s[0] + xs[1] * w2s[1])
                  + (xs[2] * w2s[2] + xs[3] * w2s[3]))
            t2 = ((xs[4] * w2s[4] + xs[5] * w2s[5])
                  + (xs[6] * w2s[6] + xs[7] * w2s[7]))
            acc1[q, pl.ds(c, LANES)] = acc1[q, pl.ds(c, LANES)] + (s1 + t1)
            acc2[q, pl.ds(c, LANES)] = acc2[q, pl.ds(c, LANES)] + (s2 + t2)

    def _pair(gg, _):
        w1v = wv1[gg >> 3, pl.ds((gg & 7) * LANES, LANES)]
        w2v = wv2[gg >> 3, pl.ds((gg & 7) * LANES, LANES)]
        for d in range(2):
            g = gg * 2 + d
            w1s = [jnp.broadcast_to(w1v[d * CL + li], (LANES,))
                   for li in range(CL)]
            w2s = [jnp.broadcast_to(w2v[d * CL + li], (LANES,))
                   for li in range(CL)]

            @pl.when(g + 1 < NCHUNK)
            def _start_next():
                _chunk_copy(g + 1, 1 - d).start()

            _chunk_copy(g, d).wait()
            _compute(d, w1s, w2s)
        return 0

    lax.fori_loop(0, NCHUNK // 2, _pair, 0)

    out_row = (b - BT) * R + r0
    pltpu.sync_copy(acc1, a1_hbm.at[pl.ds(out_row, RW)])
    pltpu.sync_copy(acc2, a2_hbm.at[pl.ds(out_row, RW)])


@jax.jit
def _sc_reduce(x4, w1, w2):
    mesh = plsc.VectorSubcoreMesh(core_axis_name="c", subcore_axis_name="s",
                                  num_cores=NC, num_subcores=NS)
    return pl.kernel(
        _sc_body,
        out_type=(jax.ShapeDtypeStruct((NSC * R, 128), jnp.float32),
                  jax.ShapeDtypeStruct((NSC * R, 128), jnp.float32)),
        mesh=mesh,
        scratch_types=(
            pltpu.VMEM((2, CL, RW, 128), jnp.float32),
            pltpu.VMEM((2, 128), jnp.float32),
            pltpu.VMEM((2, 128), jnp.float32),
            pltpu.VMEM((RW, 128), jnp.float32),
            pltpu.VMEM((RW, 128), jnp.float32),
            pltpu.SemaphoreType.DMA,
            pltpu.SemaphoreType.DMA,
        ),
        name="cons_net_sc_reduce",
    )(x4, w1, w2)


CLT = 256


def _tcr_body(w1, w2, x, y):
    b0 = pl.program_id(0) * TBB
    lc = pl.program_id(1)
    l0 = lc * CLT

    zero = jnp.zeros((R, F), jnp.float32)
    init = []
    for j in range(TBB):
        init.append(jnp.where(lc > 0, y[j, 0], zero))
        init.append(jnp.where(lc > 0, y[j, 1], zero))

    def _lbody(l, accs):
        new = []
        for j in range(TBB):
            xl = x[j, l]
            new.append(accs[2 * j] + w1[b0 + j, l0 + l] * xl)
            new.append(accs[2 * j + 1] + w2[b0 + j, l0 + l] * xl)
        return tuple(new)

    accs = lax.fori_loop(0, CLT, _lbody, tuple(init), unroll=4)
    for j in range(TBB):
        y[j, 0] = accs[2 * j]
        y[j, 1] = accs[2 * j + 1]


@jax.jit
def _tc_reduce(x4, w1, w2):
    smem_full = pl.BlockSpec((B, L), lambda i, lc: (0, 0),
                             memory_space=pltpu.SMEM)
    return pl.pallas_call(
        _tcr_body,
        grid=(BT // TBB, L // CLT),
        in_specs=[smem_full, smem_full,
                  pl.BlockSpec((TBB, CLT, R, F), lambda i, lc: (i, lc, 0, 0))],
        out_specs=pl.BlockSpec((TBB, 2, R, F), lambda i, lc: (i, 0, 0, 0)),
        out_shape=jax.ShapeDtypeStruct((BT, 2, R, F), jnp.float32),
        name="cons_net_tc_reduce",
    )(w1, w2, x4)


def _tc_body(y2, a1s, a2s, cl, cr, rf, rr, w1, w2, out, m1, m2):
    clv = cl[...]
    crv = cr[...]
    rrv = rr[...]

    def _bt(b, _):
        base = b * 2 * R
        acc = jnp.dot(clv, y2[pl.ds(base, R), :],
                      preferred_element_type=jnp.float32)
        acc = acc + jnp.dot(crv, y2[pl.ds(base + R, R), :],
                            preferred_element_type=jnp.float32)
        out[pl.ds(b * R, R), :] = acc + rrv * rf[pl.ds(b, 1), :]
        return 0

    def _bs(b, _):
        base = (b - BT) * R
        acc = jnp.dot(clv, a1s[pl.ds(base, R), :],
                      preferred_element_type=jnp.float32)
        acc = acc + jnp.dot(crv, a2s[pl.ds(base, R), :],
                            preferred_element_type=jnp.float32)
        out[pl.ds(b * R, R), :] = acc + rrv * rf[pl.ds(b, 1), :]
        return 0

    if BT:
        lax.fori_loop(0, BT, _bt, 0)
    if NSC:
        lax.fori_loop(BT, B, _bs, 0)
    m1[...] = jnp.max(w1[...], axis=1, keepdims=True)
    m2[...] = jnp.max(w2[...], axis=1, keepdims=True)


@jax.jit
def _tc_epilogue(y2, a1s, a2s, cl, cr, rf, rr, w1, w2):
    return pl.pallas_call(
        _tc_body,
        out_shape=(jax.ShapeDtypeStruct((B * R, F), jnp.float32),
                   jax.ShapeDtypeStruct((B, 1), jnp.float32),
                   jax.ShapeDtypeStruct((B, 1), jnp.float32)),
        name="cons_net_tc_epilogue",
    )(y2, a1s, a2s, cl, cr, rf, rr, w1, w2)


def kernel(x, arg1_weight, arg2_weight, root_filler, cons_l, cons_r, root_role):
    x4 = x.transpose(0, 1, 3, 2)
    x3 = x4.reshape(B, L, FR)
    w1_2d = arg1_weight.reshape(B * L // 128, 128)
    w2_2d = arg2_weight.reshape(B * L // 128, 128)

    if NSC:
        a1s, a2s = _sc_reduce(x4, w1_2d, w2_2d)
    else:
        a1s = a2s = jnp.zeros((1 * R, 128), jnp.float32)
    if BT:
        y = _tc_reduce(x4, arg1_weight, arg2_weight)
        y2 = y.reshape(BT * 2 * R, F)
    else:
        y2 = jnp.zeros((1 * 2 * R, F), jnp.float32)

    out_brf, m1, m2 = _tc_epilogue(
        y2, a1s, a2s, cons_l, cons_r,
        root_filler, root_role.reshape(R, 1),
        arg1_weight, arg2_weight)
    return (out_brf.reshape(B, R, F).transpose(0, 2, 1),
            m1.reshape(B), m2.reshape(B))
